# Initial kernel scaffold; baseline (speedup 1.0000x reference)
import jax
import jax.numpy as jnp
from jax.experimental import pallas as pl


def kernel(x, pre_w, pre_b, in_proj_w, conv_w, conv_b, x_proj_w, dt_w, dt_b, A_log, D, out_proj_w, norm_g, norm_b, cls_w, cls_b):
    raise NotImplementedError("write your pallas kernel here")


# speedup vs baseline: 4.1079x; 4.1079x over previous
import jax
import jax.numpy as jnp
from jax.experimental import pallas as pl
from jax.experimental.pallas import tpu as pltpu

_D_MODEL = 256
_D_INNER = 512
_DT_RANK = 16
_D_STATE = 16
_N_BLOCKS = 5
_LN_EPS = 1e-5
_BB = 512
_SEC = 128


def _silu(v):
    return v * jax.nn.sigmoid(v)


def _body(x_ref, pre_wT_ref, pre_b_ref, in_wT_ref, cb_ref, xp_wT_ref,
          dt_wT_ref, dt_b_ref, d_ref, out_wT_ref, g_ref, b_ref,
          cls_wT_ref, cls_b_ref, o_ref):
    f32 = jnp.float32
    h = jnp.dot(x_ref[...], pre_wT_ref[...],
                preferred_element_type=f32) + pre_b_ref[...]
    for i in range(_N_BLOCKS):
        xz = jnp.dot(h, in_wT_ref[i], preferred_element_type=f32)
        xs = _silu(xz[:, :_D_INNER] + cb_ref[i])
        z = xz[:, _D_INNER:]
        xdb = jnp.dot(xs, xp_wT_ref[i], preferred_element_type=f32)
        bc = jnp.sum(xdb[:, _SEC:2 * _SEC] * xdb[:, 2 * _SEC:],
                     axis=1, keepdims=True)
        dt = jax.nn.softplus(
            jnp.dot(xdb[:, :_SEC], dt_wT_ref[i],
                    preferred_element_type=f32) + dt_b_ref[i])
        y = xs * (dt * bc + d_ref[i]) * _silu(z)
        h = h + jnp.dot(y, out_wT_ref[i], preferred_element_type=f32)
    mu = jnp.mean(h, axis=1, keepdims=True)
    hc = h - mu
    var = jnp.mean(hc * hc, axis=1, keepdims=True)
    hn = hc * jax.lax.rsqrt(var + _LN_EPS) * g_ref[...] + b_ref[...]
    o_ref[...] = jnp.dot(hn, cls_wT_ref[...],
                         preferred_element_type=f32) + cls_b_ref[...]


def kernel(x, pre_w, pre_b, in_proj_w, conv_w, conv_b, x_proj_w, dt_w,
           dt_b, A_log, D, out_proj_w, norm_g, norm_b, cls_w, cls_b):
    del A_log
    batch = x.shape[0]
    f32 = jnp.float32

    pre_wT = pre_w.T
    tap = conv_w[:, :, 0, -1]
    scale = jnp.concatenate([tap, jnp.ones_like(tap)], axis=1)
    in_wT = in_proj_w.transpose(0, 2, 1) * scale[:, None, :]
    xp_wT = x_proj_w.transpose(0, 2, 1)
    pad = ((0, 0), (0, 0), (0, _SEC - _DT_RANK))
    xp_wT_pad = jnp.concatenate(
        [jnp.pad(xp_wT[..., :_DT_RANK], pad),
         jnp.pad(xp_wT[..., _DT_RANK:_DT_RANK + _D_STATE], pad),
         jnp.pad(xp_wT[..., _DT_RANK + _D_STATE:], pad)], axis=2)
    dt_wT_pad = jnp.pad(dt_w.transpose(0, 2, 1),
                        ((0, 0), (0, _SEC - _DT_RANK), (0, 0)))
    cb = conv_b[:, None, :]
    dtb = dt_b[:, None, :]
    dd = D[:, None, :]
    cls_wT = cls_w.T
    n_cls = cls_w.shape[0]

    inv = lambda *blk: pl.BlockSpec(blk, lambda i: (0,) * len(blk))
    grid = (batch // _BB,)
    out = pl.pallas_call(
        _body,
        grid=grid,
        in_specs=[
            pl.BlockSpec((_BB, x.shape[1]), lambda i: (i, 0)),
            inv(*pre_wT.shape),
            inv(1, _D_MODEL),
            inv(_N_BLOCKS, _D_MODEL, 2 * _D_INNER),
            inv(_N_BLOCKS, 1, _D_INNER),
            inv(_N_BLOCKS, _D_INNER, 3 * _SEC),
            inv(_N_BLOCKS, _SEC, _D_INNER),
            inv(_N_BLOCKS, 1, _D_INNER),
            inv(_N_BLOCKS, 1, _D_INNER),
            inv(_N_BLOCKS, _D_INNER, _D_MODEL),
            inv(1, _D_MODEL),
            inv(1, _D_MODEL),
            inv(_D_MODEL, n_cls),
            inv(1, n_cls),
        ],
        out_specs=pl.BlockSpec((_BB, n_cls), lambda i: (i, 0)),
        out_shape=jax.ShapeDtypeStruct((batch, n_cls), f32),
        compiler_params=pltpu.CompilerParams(
            dimension_semantics=("parallel",),
            vmem_limit_bytes=100 * 1024 * 1024,
        ),
    )(x, pre_wT, pre_b[None, :], in_wT, cb, xp_wT_pad, dt_wT_pad, dtb,
      dd, out_proj_w.transpose(0, 2, 1), norm_g[None, :],
      norm_b[None, :], cls_wT, cls_b[None, :])
    return out


# speedup vs baseline: 4.2684x; 1.0391x over previous
import jax
import jax.numpy as jnp
from jax.experimental import pallas as pl
from jax.experimental.pallas import tpu as pltpu

_D_MODEL = 256
_D_INNER = 512
_DT_RANK = 16
_D_STATE = 16
_N_BLOCKS = 5
_LN_EPS = 1e-5
_BB = 512
_SEC = 128


def _silu(v):
    return v * jax.nn.sigmoid(v)


def _body(x_ref, pre_wT_ref, pre_b_ref, in_wT_ref, cb_ref, xp_wT_ref,
          dt_wT_ref, dt_b_ref, d_ref, out_wT_ref, g_ref, b_ref,
          cls_wT_ref, cls_b_ref, o_ref):
    f32 = jnp.float32
    bf16 = jnp.bfloat16
    h = jnp.dot(x_ref[...].astype(bf16), pre_wT_ref[...],
                preferred_element_type=f32) + pre_b_ref[...]
    for i in range(_N_BLOCKS):
        xz = jnp.dot(h.astype(bf16), in_wT_ref[i],
                     preferred_element_type=f32)
        xs = _silu(xz[:, :_D_INNER] + cb_ref[i])
        z = xz[:, _D_INNER:]
        xdb = jnp.dot(xs.astype(bf16), xp_wT_ref[i],
                      preferred_element_type=f32)
        bc = jnp.sum(xdb[:, _SEC:2 * _SEC] * xdb[:, 2 * _SEC:],
                     axis=1, keepdims=True)
        dt = jax.nn.softplus(
            jnp.dot(xdb[:, :_SEC].astype(bf16), dt_wT_ref[i],
                    preferred_element_type=f32) + dt_b_ref[i])
        y = xs * (dt * bc + d_ref[i]) * _silu(z)
        h = h + jnp.dot(y.astype(bf16), out_wT_ref[i],
                        preferred_element_type=f32)
    mu = jnp.mean(h, axis=1, keepdims=True)
    hc = h - mu
    var = jnp.mean(hc * hc, axis=1, keepdims=True)
    hn = hc * jax.lax.rsqrt(var + _LN_EPS) * g_ref[...] + b_ref[...]
    o_ref[...] = jnp.dot(hn, cls_wT_ref[...],
                         preferred_element_type=f32) + cls_b_ref[...]


def kernel(x, pre_w, pre_b, in_proj_w, conv_w, conv_b, x_proj_w, dt_w,
           dt_b, A_log, D, out_proj_w, norm_g, norm_b, cls_w, cls_b):
    del A_log
    batch = x.shape[0]
    f32 = jnp.float32

    pre_wT = pre_w.T
    tap = conv_w[:, :, 0, -1]
    scale = jnp.concatenate([tap, jnp.ones_like(tap)], axis=1)
    in_wT = in_proj_w.transpose(0, 2, 1) * scale[:, None, :]
    xp_wT = x_proj_w.transpose(0, 2, 1)
    pad = ((0, 0), (0, 0), (0, _SEC - _DT_RANK))
    xp_wT_pad = jnp.concatenate(
        [jnp.pad(xp_wT[..., :_DT_RANK], pad),
         jnp.pad(xp_wT[..., _DT_RANK:_DT_RANK + _D_STATE], pad),
         jnp.pad(xp_wT[..., _DT_RANK + _D_STATE:], pad)], axis=2)
    dt_wT_pad = jnp.pad(dt_w.transpose(0, 2, 1),
                        ((0, 0), (0, _SEC - _DT_RANK), (0, 0)))
    cb = conv_b[:, None, :]
    dtb = dt_b[:, None, :]
    dd = D[:, None, :]
    cls_wT = cls_w.T
    n_cls = cls_w.shape[0]
    bf16 = jnp.bfloat16
    pre_wT = pre_wT.astype(bf16)
    in_wT = in_wT.astype(bf16)
    xp_wT_pad = xp_wT_pad.astype(bf16)
    dt_wT_pad = dt_wT_pad.astype(bf16)
    out_wT = out_proj_w.transpose(0, 2, 1).astype(bf16)

    inv = lambda *blk: pl.BlockSpec(blk, lambda i: (0,) * len(blk))
    grid = (batch // _BB,)
    out = pl.pallas_call(
        _body,
        grid=grid,
        in_specs=[
            pl.BlockSpec((_BB, x.shape[1]), lambda i: (i, 0)),
            inv(*pre_wT.shape),
            inv(1, _D_MODEL),
            inv(_N_BLOCKS, _D_MODEL, 2 * _D_INNER),
            inv(_N_BLOCKS, 1, _D_INNER),
            inv(_N_BLOCKS, _D_INNER, 3 * _SEC),
            inv(_N_BLOCKS, _SEC, _D_INNER),
            inv(_N_BLOCKS, 1, _D_INNER),
            inv(_N_BLOCKS, 1, _D_INNER),
            inv(_N_BLOCKS, _D_INNER, _D_MODEL),
            inv(1, _D_MODEL),
            inv(1, _D_MODEL),
            inv(_D_MODEL, n_cls),
            inv(1, n_cls),
        ],
        out_specs=pl.BlockSpec((_BB, n_cls), lambda i: (i, 0)),
        out_shape=jax.ShapeDtypeStruct((batch, n_cls), f32),
        compiler_params=pltpu.CompilerParams(
            dimension_semantics=("parallel",),
            vmem_limit_bytes=100 * 1024 * 1024,
        ),
    )(x, pre_wT, pre_b[None, :], in_wT, cb, xp_wT_pad, dt_wT_pad, dtb,
      dd, out_wT, norm_g[None, :],
      norm_b[None, :], cls_wT, cls_b[None, :])
    return out
